# baseline (speedup 1.0000x reference)
import functools

import jax
import jax.numpy as jnp
from jax import lax
from jax.experimental import pallas as pl
from jax.experimental.pallas import tpu as pltpu
from jax.experimental.pallas import tpu_sc as plsc


def _sc_body(dsp_hbm, hsp_hbm, wpt_hbm, out_hbm,
             dspv, hspv, wv, hwv, rows, sem0, sem1):
    wid = lax.axis_index("s") * 2 + lax.axis_index("c")
    ch0 = wid * 8
    pltpu.sync_copy(dsp_hbm.at[pl.ds(ch0, 8)], dspv)
    pltpu.sync_copy(hsp_hbm.at[pl.ds(ch0, 8)], hspv)
    pltpu.sync_copy(wpt_hbm.at[pl.ds(ch0, 8)], wv)
    sems = (sem0, sem1)
    pending = {0: [], 1: []}

    for cc in range(8):
        buf = cc % 2
        w0 = wv[cc, 0:16]
        w1 = wv[cc, 16:32]

        def build_hw(j, _):
            hj = hspv[cc, pl.ds(j * 16, 16)]
            hwv[pl.ds(j * 32, 16)] = hj + w0
            hwv[pl.ds(j * 32 + 16, 16)] = hj + w1
            return 0

        lax.fori_loop(0, 32, build_hw, 0, unroll=4)

        for cp in pending[buf]:
            cp.wait()
        pending[buf] = []

        def build_row(i, _):
            di = dspv[cc, pl.ds(i * 16, 16)]

            def inner(m, _):
                rows[buf, pl.ds(i * 1024 + m * 16, 16)] = (
                    hwv[pl.ds(m * 16, 16)] + di)
                return 0

            lax.fori_loop(0, 64, inner, 0, unroll=8)
            return 0

        lax.fori_loop(0, 32, build_row, 0)

        r = ch0 + cc
        cp0 = pltpu.make_async_copy(rows.at[buf], out_hbm.at[r], sems[buf])
        cp0.start()
        cp1 = pltpu.make_async_copy(rows.at[buf], out_hbm.at[r + 256],
                                    sems[buf])
        cp1.start()
        pending[buf] += [cp0, cp1]

    for buf in range(2):
        for cp in pending[buf]:
            cp.wait()


def kernel(x, d_weight, h_weight, w_weight):
    B = x.shape[0]
    d, h, w = x.shape[-3:]
    c = d_weight.shape[1]
    C = 256

    f32 = jnp.float32
    dpt = jnp.zeros((C, d), f32).at[0:c, :].set(d_weight[:d].T.astype(f32))
    hpt = jnp.zeros((C, h), f32).at[c:2 * c, :].set(h_weight[:h].T.astype(f32))
    wpt = jnp.zeros((C, w), f32).at[2 * c:C, :].set(
        w_weight[:w, : C - 2 * c].T.astype(f32))
    dsp = jnp.repeat(dpt, 16, axis=1)
    hsp = jnp.repeat(hpt, 16, axis=1)

    S = d * h * w
    mesh = plsc.VectorSubcoreMesh(core_axis_name="c", subcore_axis_name="s")
    sc = functools.partial(
        pl.kernel,
        mesh=mesh,
        out_type=jax.ShapeDtypeStruct((B * C, S), f32),
        scratch_types=[
            pltpu.VMEM((8, 16 * d), f32),
            pltpu.VMEM((8, 16 * h), f32),
            pltpu.VMEM((8, w), f32),
            pltpu.VMEM((h * w,), f32),
            pltpu.VMEM((2, S), f32),
            pltpu.SemaphoreType.DMA,
            pltpu.SemaphoreType.DMA,
        ],
    )(_sc_body)
    out2 = sc(dsp, hsp, wpt)
    return out2.reshape(B, C, d, h, w)


# speedup vs baseline: 1.2677x; 1.2677x over previous
import functools

import jax
import jax.numpy as jnp
from jax import lax
from jax.experimental import pallas as pl
from jax.experimental.pallas import tpu as pltpu
from jax.experimental.pallas import tpu_sc as plsc


def _sc_body(dsp_hbm, hsp_hbm, wpt_hbm, out_hbm,
             dspv, hspv, wv, hwv, rows, sem0, sem1):
    wid = lax.axis_index("s") * 2 + lax.axis_index("c")
    ch0 = wid * 8
    pltpu.sync_copy(dsp_hbm.at[pl.ds(ch0, 8)], dspv)
    pltpu.sync_copy(hsp_hbm.at[pl.ds(ch0, 8)], hspv)
    pltpu.sync_copy(wpt_hbm.at[pl.ds(ch0, 8)], wv)
    sems = (sem0, sem1)
    pending = {0: [], 1: []}

    for cc in range(8):
        buf = cc % 2
        w0 = wv[cc, 0:16]
        w1 = wv[cc, 16:32]

        @plsc.parallel_loop(0, 32, unroll=4)
        def _build_hw(j):
            hj = hspv[cc, pl.ds(j * 16, 16)]
            hwv[pl.ds(j * 32, 16)] = hj + w0
            hwv[pl.ds(j * 32 + 16, 16)] = hj + w1

        for cp in pending[buf]:
            cp.wait()
        pending[buf] = []

        def build_row(i, _):
            di = dspv[cc, pl.ds(i * 16, 16)]

            @plsc.parallel_loop(0, 64, unroll=8)
            def _inner(m):
                rows[buf, pl.ds(i * 1024 + m * 16, 16)] = (
                    hwv[pl.ds(m * 16, 16)] + di)

            return 0

        lax.fori_loop(0, 32, build_row, 0)

        r = ch0 + cc
        cp0 = pltpu.make_async_copy(rows.at[buf], out_hbm.at[r], sems[buf])
        cp0.start()
        cp1 = pltpu.make_async_copy(rows.at[buf], out_hbm.at[r + 256],
                                    sems[buf])
        cp1.start()
        pending[buf] += [cp0, cp1]

    for buf in range(2):
        for cp in pending[buf]:
            cp.wait()


def kernel(x, d_weight, h_weight, w_weight):
    B = x.shape[0]
    d, h, w = x.shape[-3:]
    c = d_weight.shape[1]
    C = 256

    f32 = jnp.float32
    dpt = jnp.zeros((C, d), f32).at[0:c, :].set(d_weight[:d].T.astype(f32))
    hpt = jnp.zeros((C, h), f32).at[c:2 * c, :].set(h_weight[:h].T.astype(f32))
    wpt = jnp.zeros((C, w), f32).at[2 * c:C, :].set(
        w_weight[:w, : C - 2 * c].T.astype(f32))
    dsp = jnp.repeat(dpt, 16, axis=1)
    hsp = jnp.repeat(hpt, 16, axis=1)

    S = d * h * w
    mesh = plsc.VectorSubcoreMesh(core_axis_name="c", subcore_axis_name="s")
    sc = functools.partial(
        pl.kernel,
        mesh=mesh,
        out_type=jax.ShapeDtypeStruct((B * C, S), f32),
        scratch_types=[
            pltpu.VMEM((8, 16 * d), f32),
            pltpu.VMEM((8, 16 * h), f32),
            pltpu.VMEM((8, w), f32),
            pltpu.VMEM((h * w,), f32),
            pltpu.VMEM((2, S), f32),
            pltpu.SemaphoreType.DMA,
            pltpu.SemaphoreType.DMA,
        ],
    )(_sc_body)
    out2 = sc(dsp, hsp, wpt)
    return out2.reshape(B, C, d, h, w)


# speedup vs baseline: 2.8956x; 2.2842x over previous
import functools

import jax
import jax.numpy as jnp
from jax import lax
from jax.experimental import pallas as pl
from jax.experimental.pallas import tpu as pltpu
from jax.experimental.pallas import tpu_sc as plsc


def _sc_body(dsp_hbm, hsp_hbm, wpt_hbm, out_hbm,
             dspv, hspv, wv, hwv, rows, sem0, sem1):
    wid = lax.axis_index("s") * 2 + lax.axis_index("c")
    ch0 = wid * 8
    pltpu.sync_copy(dsp_hbm.at[pl.ds(ch0, 8)], dspv)
    pltpu.sync_copy(hsp_hbm.at[pl.ds(ch0, 8)], hspv)
    pltpu.sync_copy(wpt_hbm.at[pl.ds(ch0, 8)], wv)
    sems = (sem0, sem1)
    pending = {0: [], 1: []}

    for cc in range(8):
        buf = cc % 2
        w0 = wv[cc, 0:16]
        w1 = wv[cc, 16:32]

        @plsc.parallel_loop(0, 32, unroll=4)
        def _build_hw(j):
            hj = hspv[cc, pl.ds(j * 16, 16)]
            hwv[pl.ds(j * 32, 16)] = hj + w0
            hwv[pl.ds(j * 32 + 16, 16)] = hj + w1

        for cp in pending[buf]:
            cp.wait()
        pending[buf] = []

        def build_row(i, _):
            di = dspv[cc, pl.ds(i * 16, 16)]

            @plsc.parallel_loop(0, 64, unroll=8)
            def _inner(m):
                rows[buf, i, pl.ds(m * 16, 16)] = hwv[pl.ds(m * 16, 16)] + di

            return 0

        lax.fori_loop(0, 32, build_row, 0)

        r = ch0 + cc
        cp0 = pltpu.make_async_copy(rows.at[buf], out_hbm.at[0, r], sems[buf])
        cp0.start()
        cp1 = pltpu.make_async_copy(rows.at[buf], out_hbm.at[1, r], sems[buf])
        cp1.start()
        pending[buf] += [cp0, cp1]

    for buf in range(2):
        for cp in pending[buf]:
            cp.wait()


def kernel(x, d_weight, h_weight, w_weight):
    B = x.shape[0]
    d, h, w = x.shape[-3:]
    c = d_weight.shape[1]
    C = 256

    f32 = jnp.float32
    dpt = jnp.zeros((C, d), f32).at[0:c, :].set(d_weight[:d].T.astype(f32))
    hpt = jnp.zeros((C, h), f32).at[c:2 * c, :].set(h_weight[:h].T.astype(f32))
    wpt = jnp.zeros((C, w), f32).at[2 * c:C, :].set(
        w_weight[:w, : C - 2 * c].T.astype(f32))
    dsp = jnp.repeat(dpt, 16, axis=1)
    hsp = jnp.repeat(hpt, 16, axis=1)

    mesh = plsc.VectorSubcoreMesh(core_axis_name="c", subcore_axis_name="s")
    sc = functools.partial(
        pl.kernel,
        mesh=mesh,
        out_type=jax.ShapeDtypeStruct((B, C, d, h * w), f32),
        scratch_types=[
            pltpu.VMEM((8, 16 * d), f32),
            pltpu.VMEM((8, 16 * h), f32),
            pltpu.VMEM((8, w), f32),
            pltpu.VMEM((h * w,), f32),
            pltpu.VMEM((2, d, h * w), f32),
            pltpu.SemaphoreType.DMA,
            pltpu.SemaphoreType.DMA,
        ],
    )(_sc_body)
    out4 = sc(dsp, hsp, wpt)
    return out4.reshape(B, C, d, h, w)


# speedup vs baseline: 3.2496x; 1.1223x over previous
import jax
import jax.numpy as jnp
from jax.experimental import pallas as pl


def _body(dpt_ref, hpt_ref, wpt_ref, out_ref):
    nb = out_ref.shape[0]
    cb, d = dpt_ref.shape
    hw = hpt_ref.shape[1] * wpt_ref.shape[1]
    h = hpt_ref[...]
    w = wpt_ref[...]
    hwsum = (h[:, :, None] + w[:, None, :]).reshape(cb, hw)
    dv = dpt_ref[...]
    for i in range(d):
        row = hwsum + dv[:, i][:, None]
        for b in range(nb):
            out_ref[b, :, i, :] = row


def kernel(x, d_weight, h_weight, w_weight):
    B = x.shape[0]
    d, h, w = x.shape[-3:]
    c = d_weight.shape[1]
    C = 256

    f32 = jnp.float32
    dpt = jnp.zeros((C, d), f32).at[0:c, :].set(d_weight[:d].T.astype(f32))
    hpt = jnp.zeros((C, h), f32).at[c:2 * c, :].set(h_weight[:h].T.astype(f32))
    wpt = jnp.zeros((C, w), f32).at[2 * c:C, :].set(
        w_weight[:w, : C - 2 * c].T.astype(f32))

    CB = 64
    grid = (C // CB,)
    out4 = pl.pallas_call(
        _body,
        grid=grid,
        in_specs=[
            pl.BlockSpec((CB, d), lambda pc: (pc, 0)),
            pl.BlockSpec((CB, h), lambda pc: (pc, 0)),
            pl.BlockSpec((CB, w), lambda pc: (pc, 0)),
        ],
        out_specs=pl.BlockSpec((B, CB, d, h * w), lambda pc: (0, pc, 0, 0)),
        out_shape=jax.ShapeDtypeStruct((B, C, d, h * w), f32),
    )(dpt, hpt, wpt)
    return out4.reshape(B, C, d, h, w)


# speedup vs baseline: 3.4201x; 1.0524x over previous
import jax
import jax.numpy as jnp
from jax import lax
from jax.experimental import pallas as pl
from jax.experimental.pallas import tpu as pltpu

_CB = 16
_NBUF = 4


def _body(dpt_ref, hpt_ref, wpt_ref, out_hbm, vbuf, sems):
    ncs = pl.num_programs(0)
    pc = pl.program_id(0)
    slot = lax.rem(pc, _NBUF)
    nb = out_hbm.shape[0]
    cb, d = dpt_ref.shape
    hw = out_hbm.shape[3]

    @pl.when(pc >= _NBUF)
    def _():
        pltpu.make_async_copy(
            vbuf.at[slot], out_hbm.at[:, pl.ds(0, _CB)], sems.at[slot]
        ).wait()

    h = hpt_ref[pl.ds(pc * _CB, _CB), :]
    w = wpt_ref[pl.ds(pc * _CB, _CB), :]
    hwsum = (h[:, :, None] + w[:, None, :]).reshape(_CB, hw)
    dv = dpt_ref[pl.ds(pc * _CB, _CB), :]
    for i in range(d):
        row = hwsum + dv[:, i][:, None]
        for b in range(nb):
            vbuf[slot, b, :, i, :] = row

    pltpu.make_async_copy(
        vbuf.at[slot], out_hbm.at[:, pl.ds(pc * _CB, _CB)], sems.at[slot]
    ).start()

    @pl.when(pc == ncs - 1)
    def _():
        for s in range(_NBUF):
            pltpu.make_async_copy(
                vbuf.at[s], out_hbm.at[:, pl.ds(0, _CB)], sems.at[s]
            ).wait()


def kernel(x, d_weight, h_weight, w_weight):
    B = x.shape[0]
    d, h, w = x.shape[-3:]
    c = d_weight.shape[1]
    C = 256

    f32 = jnp.float32
    dpt = jnp.zeros((C, d), f32).at[0:c, :].set(d_weight[:d].T.astype(f32))
    hpt = jnp.zeros((C, h), f32).at[c:2 * c, :].set(h_weight[:h].T.astype(f32))
    wpt = jnp.zeros((C, w), f32).at[2 * c:C, :].set(
        w_weight[:w, : C - 2 * c].T.astype(f32))

    grid = (C // _CB,)
    out4 = pl.pallas_call(
        _body,
        grid=grid,
        in_specs=[
            pl.BlockSpec((C, d), lambda pc: (0, 0)),
            pl.BlockSpec((C, h), lambda pc: (0, 0)),
            pl.BlockSpec((C, w), lambda pc: (0, 0)),
        ],
        out_specs=pl.BlockSpec(memory_space=pltpu.HBM),
        out_shape=jax.ShapeDtypeStruct((B, C, d, h * w), f32),
        scratch_shapes=[
            pltpu.VMEM((_NBUF, B, _CB, d, h * w), f32),
            pltpu.SemaphoreType.DMA((_NBUF,)),
        ],
    )(dpt, hpt, wpt)
    return out4.reshape(B, C, d, h, w)
